# baseline (speedup 1.0000x reference)
import functools

import jax
import jax.numpy as jnp
from jax import lax
from jax.experimental import pallas as pl
from jax.experimental.pallas import tpu as pltpu
from jax.experimental.pallas import tpu_sc as plsc

N = 10000
E = 320000
NFEAT = 128
NHID = 64
H = 2 * NHID
MAX_LOGSTD = 10.0

NC = 2
NS = 16
NW = NC * NS
CHUNK = 64
NCH = 160
TPE = NCH * CHUNK
EPAD = NW * TPE
PH = 4
CPP = NCH // PH
NACC = 10240
RPT = NACC // NS
NB = 4
BLK = 128
NBLK = NACC // BLK

_mesh = plsc.VectorSubcoreMesh(core_axis_name="c", subcore_axis_name="s")


@functools.partial(
    pl.kernel,
    out_type=jax.ShapeDtypeStruct((NW, 1, NACC), jnp.float32),
    mesh=_mesh,
    compiler_params=pltpu.CompilerParams(needs_layout_passes=False),
    scratch_types=[
        pltpu.VMEM((NACC,), jnp.float32),
        pltpu.VMEM((NCH, CHUNK), jnp.int32),
    ],
)
def _deg_kernel(dst_hbm, zeros_hbm, out_hbm, hist, didx):
    c = lax.axis_index("c")
    s = lax.axis_index("s")
    wid = s * NC + c
    pltpu.sync_copy(zeros_hbm, hist)
    pltpu.sync_copy(dst_hbm.at[wid], didx)
    ones = jnp.full((16,), 1.0, jnp.float32)

    def body(j, carry):
        for k in range(CHUNK // 16):
            idx = didx[j, pl.ds(k * 16, 16)]
            plsc.addupdate_scatter(hist, [idx], ones)
        return carry

    lax.fori_loop(0, NCH, body, 0)
    pltpu.sync_copy(hist, out_hbm.at[wid, 0])


@functools.partial(
    pl.kernel,
    out_type=jax.ShapeDtypeStruct((NC, NACC, H), jnp.float32),
    mesh=_mesh,
    scratch_types=[
        pltpu.VMEM_SHARED((NACC, H), jnp.float32),
        pltpu.VMEM((CPP, CHUNK), jnp.int32),
        pltpu.VMEM((CPP, CHUNK), jnp.int32),
        pltpu.VMEM((NB, CHUNK, H), jnp.float32),
        pltpu.SemaphoreType.DMA,
        pltpu.SemaphoreType.DMA,
        pltpu.SemaphoreType.DMA,
        pltpu.SemaphoreType.DMA,
    ],
)
def _adj_kernel(y_hbm, src_hbm, dst_hbm, zeros_hbm, out_hbm,
                acc, sidx, didx, rows, sem0, sem1, sem2, sem3):
    sems = (sem0, sem1, sem2, sem3)
    c = lax.axis_index("c")
    s = lax.axis_index("s")
    wid = s * NC + c
    pltpu.sync_copy(zeros_hbm, acc.at[pl.ds(s * RPT, RPT)])
    plsc.subcore_barrier()

    for p in range(PH):
        pltpu.sync_copy(src_hbm.at[wid, pl.ds(p * CPP, CPP)], sidx)
        pltpu.sync_copy(dst_hbm.at[wid, pl.ds(p * CPP, CPP)], didx)

        for b in range(NB):
            pltpu.async_copy(y_hbm.at[sidx.at[b]], rows.at[b], sems[b])

        def body(k, carry):
            for b in range(NB):
                chunk = k * NB + b
                pltpu.make_async_copy(y_hbm.at[sidx.at[chunk]],
                                      rows.at[b], sems[b]).wait()
                pltpu.sync_copy(rows.at[b], acc.at[didx.at[chunk]], add=True)

                @pl.when(chunk + NB < CPP)
                def _():
                    pltpu.async_copy(y_hbm.at[sidx.at[chunk + NB]],
                                     rows.at[b], sems[b])
            return carry

        lax.fori_loop(0, CPP // NB, body, 0)

    plsc.subcore_barrier()
    pltpu.sync_copy(acc.at[pl.ds(s * RPT, RPT)],
                    out_hbm.at[c, pl.ds(s * RPT, RPT)])


def _tc1_body(x_ref, w_ref, b_ref, d_ref, y1_ref, dinv_ref):
    i = pl.program_id(0)
    ones_w = jnp.ones((NW, 1), jnp.float32)
    deg = lax.dot_general(d_ref[...], ones_w, (((0,), (0,)), ((), ())),
                          preferred_element_type=jnp.float32) + 1.0
    rows = i * BLK + lax.broadcasted_iota(jnp.int32, (BLK, 1), 0)
    dinv = jnp.where(rows < N, lax.rsqrt(deg), 0.0)
    h0 = jnp.dot(x_ref[...], w_ref[...], preferred_element_type=jnp.float32)
    y1_ref[...] = dinv * (h0 + b_ref[...])
    dinv_ref[...] = dinv


def _tc2_body(u_ref, y1_ref, dinv_ref, w_ref, b_ref, y2_ref):
    dinv = dinv_ref[...]
    s1 = dinv * (u_ref[0] + u_ref[1] + y1_ref[...])
    h = jnp.dot(s1, w_ref[...], preferred_element_type=jnp.float32)
    h = jnp.maximum(h + b_ref[...], 0.0)
    y2_ref[...] = dinv * h


def _tc3_body(u_ref, y2_ref, dinv_ref, w_ref, b_ref, eps_ref, z_ref):
    s2 = dinv_ref[...] * (u_ref[0] + u_ref[1] + y2_ref[...])
    o = jnp.dot(s2, w_ref[...], preferred_element_type=jnp.float32)
    o = o + b_ref[...]
    mu = o[:, :NHID]
    ls = jnp.minimum(o[:, NHID:], MAX_LOGSTD)
    z_ref[...] = mu + eps_ref[...] * jnp.exp(ls)


def _row_spec(width):
    return pl.BlockSpec((BLK, width), lambda i: (i, 0))


def _full_spec(shape):
    ndim = len(shape)
    return pl.BlockSpec(shape, lambda i: (0,) * ndim)


def _parts_spec(width):
    return pl.BlockSpec((NC, BLK, width), lambda i: (0, i, 0))


def kernel(x, edge_index, lin_W, lin_b, W1, b1, Wmu, bmu, Wls, bls, eps):
    src = edge_index[0]
    dst = edge_index[1]
    pad = jnp.full((EPAD - E,), N, dtype=jnp.int32)
    src_p = jnp.concatenate([src, pad]).reshape(NW, NCH, CHUNK)
    dst_p = jnp.concatenate([dst, pad]).reshape(NW, NCH, CHUNK)
    x_p = jnp.pad(x, ((0, NACC - N), (0, 0)))
    eps_p = jnp.pad(eps, ((0, NACC - N), (0, 0)))
    zeros_n = jnp.zeros((NACC,), jnp.float32)
    zrows = jnp.zeros((RPT, H), jnp.float32)
    W2 = jnp.concatenate([Wmu, Wls], axis=1)
    b2 = jnp.concatenate([bmu, bls])[None, :]
    b1r = b1[None, :]
    linbr = lin_b[None, :]

    deg_parts = _deg_kernel(dst_p, zeros_n).reshape(NW, NACC)

    y1, dinv = pl.pallas_call(
        _tc1_body,
        grid=(NBLK,),
        in_specs=[_row_spec(NFEAT), _full_spec((NFEAT, H)),
                  _full_spec((1, H)), pl.BlockSpec((NW, BLK), lambda i: (0, i))],
        out_specs=[_row_spec(H), _row_spec(1)],
        out_shape=[jax.ShapeDtypeStruct((NACC, H), jnp.float32),
                   jax.ShapeDtypeStruct((NACC, 1), jnp.float32)],
    )(x_p, lin_W, linbr, deg_parts)

    u1 = _adj_kernel(y1, src_p, dst_p, zrows)

    y2 = pl.pallas_call(
        _tc2_body,
        grid=(NBLK,),
        in_specs=[_parts_spec(H), _row_spec(H), _row_spec(1),
                  _full_spec((H, H)), _full_spec((1, H))],
        out_specs=_row_spec(H),
        out_shape=jax.ShapeDtypeStruct((NACC, H), jnp.float32),
    )(u1, y1, dinv, W1, b1r)

    u2 = _adj_kernel(y2, src_p, dst_p, zrows)

    z = pl.pallas_call(
        _tc3_body,
        grid=(NBLK,),
        in_specs=[_parts_spec(H), _row_spec(H), _row_spec(1),
                  _full_spec((H, H)), _full_spec((1, H)), _row_spec(NHID)],
        out_specs=_row_spec(NHID),
        out_shape=jax.ShapeDtypeStruct((NACC, NHID), jnp.float32),
    )(u2, y2, dinv, W2, b2, eps_p)

    return z[:N]


# speedup vs baseline: 1.0407x; 1.0407x over previous
import functools

import jax
import jax.numpy as jnp
from jax import lax
from jax.experimental import pallas as pl
from jax.experimental.pallas import tpu as pltpu
from jax.experimental.pallas import tpu_sc as plsc

N = 10000
E = 320000
NFEAT = 128
NHID = 64
H = 2 * NHID
MAX_LOGSTD = 10.0

NC = 2
NS = 16
NW = NC * NS
CHUNK = 128
NCH = 80
TPE = NCH * CHUNK
EPAD = NW * TPE
PH = 2
CPP = NCH // PH
NACC = 10240
RPT = NACC // NS
NB = 2
BLK = 128
NBLK = NACC // BLK

_mesh = plsc.VectorSubcoreMesh(core_axis_name="c", subcore_axis_name="s")


@functools.partial(
    pl.kernel,
    out_type=jax.ShapeDtypeStruct((NW, 1, NACC), jnp.float32),
    mesh=_mesh,
    compiler_params=pltpu.CompilerParams(needs_layout_passes=False),
    scratch_types=[
        pltpu.VMEM((NACC,), jnp.float32),
        pltpu.VMEM((NCH, CHUNK), jnp.int32),
    ],
)
def _deg_kernel(dst_hbm, zeros_hbm, out_hbm, hist, didx):
    c = lax.axis_index("c")
    s = lax.axis_index("s")
    wid = s * NC + c
    pltpu.sync_copy(zeros_hbm, hist)
    pltpu.sync_copy(dst_hbm.at[wid], didx)
    ones = jnp.full((16,), 1.0, jnp.float32)

    def body(j, carry):
        for k in range(CHUNK // 16):
            idx = didx[j, pl.ds(k * 16, 16)]
            plsc.addupdate_scatter(hist, [idx], ones)
        return carry

    lax.fori_loop(0, NCH, body, 0)
    pltpu.sync_copy(hist, out_hbm.at[wid, 0])


@functools.partial(
    pl.kernel,
    out_type=jax.ShapeDtypeStruct((NC, NACC, H), jnp.float32),
    mesh=_mesh,
    scratch_types=[
        pltpu.VMEM_SHARED((NACC, H), jnp.float32),
        pltpu.VMEM((CPP, CHUNK), jnp.int32),
        pltpu.VMEM((CPP, CHUNK), jnp.int32),
        pltpu.VMEM((NB, CHUNK, H), jnp.float32),
        pltpu.SemaphoreType.DMA,
        pltpu.SemaphoreType.DMA,
        pltpu.SemaphoreType.DMA,
        pltpu.SemaphoreType.DMA,
    ],
)
def _adj_kernel(y_hbm, src_hbm, dst_hbm, zeros_hbm, out_hbm,
                acc, sidx, didx, rows, sem0, sem1, sem2, sem3):
    sems = (sem0, sem1, sem2, sem3)
    c = lax.axis_index("c")
    s = lax.axis_index("s")
    wid = s * NC + c
    pltpu.sync_copy(zeros_hbm, acc.at[pl.ds(s * RPT, RPT)])
    plsc.subcore_barrier()

    for p in range(PH):
        pltpu.sync_copy(src_hbm.at[wid, pl.ds(p * CPP, CPP)], sidx)
        pltpu.sync_copy(dst_hbm.at[wid, pl.ds(p * CPP, CPP)], didx)

        for b in range(NB):
            pltpu.async_copy(y_hbm.at[sidx.at[b]], rows.at[b], sems[b])

        def body(k, carry):
            for b in range(NB):
                chunk = k * NB + b
                pltpu.make_async_copy(y_hbm.at[sidx.at[chunk]],
                                      rows.at[b], sems[b]).wait()
                pltpu.sync_copy(rows.at[b], acc.at[didx.at[chunk]], add=True)

                @pl.when(chunk + NB < CPP)
                def _():
                    pltpu.async_copy(y_hbm.at[sidx.at[chunk + NB]],
                                     rows.at[b], sems[b])
            return carry

        lax.fori_loop(0, CPP // NB, body, 0)

    plsc.subcore_barrier()
    pltpu.sync_copy(acc.at[pl.ds(s * RPT, RPT)],
                    out_hbm.at[c, pl.ds(s * RPT, RPT)])


def _tc1_body(x_ref, w_ref, b_ref, d_ref, y1_ref, dinv_ref):
    i = pl.program_id(0)
    ones_w = jnp.ones((NW, 1), jnp.float32)
    deg = lax.dot_general(d_ref[...], ones_w, (((0,), (0,)), ((), ())),
                          preferred_element_type=jnp.float32) + 1.0
    rows = i * BLK + lax.broadcasted_iota(jnp.int32, (BLK, 1), 0)
    dinv = jnp.where(rows < N, lax.rsqrt(deg), 0.0)
    h0 = jnp.dot(x_ref[...], w_ref[...], preferred_element_type=jnp.float32)
    y1_ref[...] = dinv * (h0 + b_ref[...])
    dinv_ref[...] = dinv


def _tc2_body(u_ref, y1_ref, dinv_ref, w_ref, b_ref, y2_ref):
    dinv = dinv_ref[...]
    s1 = dinv * (u_ref[0] + u_ref[1] + y1_ref[...])
    h = jnp.dot(s1, w_ref[...], preferred_element_type=jnp.float32)
    h = jnp.maximum(h + b_ref[...], 0.0)
    y2_ref[...] = dinv * h


def _tc3_body(u_ref, y2_ref, dinv_ref, w_ref, b_ref, eps_ref, z_ref):
    s2 = dinv_ref[...] * (u_ref[0] + u_ref[1] + y2_ref[...])
    o = jnp.dot(s2, w_ref[...], preferred_element_type=jnp.float32)
    o = o + b_ref[...]
    mu = o[:, :NHID]
    ls = jnp.minimum(o[:, NHID:], MAX_LOGSTD)
    z_ref[...] = mu + eps_ref[...] * jnp.exp(ls)


def _row_spec(width):
    return pl.BlockSpec((BLK, width), lambda i: (i, 0))


def _full_spec(shape):
    ndim = len(shape)
    return pl.BlockSpec(shape, lambda i: (0,) * ndim)


def _parts_spec(width):
    return pl.BlockSpec((NC, BLK, width), lambda i: (0, i, 0))


def kernel(x, edge_index, lin_W, lin_b, W1, b1, Wmu, bmu, Wls, bls, eps):
    src = edge_index[0]
    dst = edge_index[1]
    pad_src = jnp.full((EPAD - E,), N, dtype=jnp.int32)
    pad_dst = N + 1 + (jnp.arange(EPAD - E, dtype=jnp.int32) % (NACC - N - 1))
    src_p = jnp.concatenate([src, pad_src]).reshape(NW, NCH, CHUNK)
    dst_p = jnp.concatenate([dst, pad_dst]).reshape(NW, NCH, CHUNK)
    x_p = jnp.pad(x, ((0, NACC - N), (0, 0)))
    eps_p = jnp.pad(eps, ((0, NACC - N), (0, 0)))
    zeros_n = jnp.zeros((NACC,), jnp.float32)
    zrows = jnp.zeros((RPT, H), jnp.float32)
    W2 = jnp.concatenate([Wmu, Wls], axis=1)
    b2 = jnp.concatenate([bmu, bls])[None, :]
    b1r = b1[None, :]
    linbr = lin_b[None, :]

    deg_parts = _deg_kernel(dst_p, zeros_n).reshape(NW, NACC)

    y1, dinv = pl.pallas_call(
        _tc1_body,
        grid=(NBLK,),
        in_specs=[_row_spec(NFEAT), _full_spec((NFEAT, H)),
                  _full_spec((1, H)), pl.BlockSpec((NW, BLK), lambda i: (0, i))],
        out_specs=[_row_spec(H), _row_spec(1)],
        out_shape=[jax.ShapeDtypeStruct((NACC, H), jnp.float32),
                   jax.ShapeDtypeStruct((NACC, 1), jnp.float32)],
    )(x_p, lin_W, linbr, deg_parts)

    u1 = _adj_kernel(y1, src_p, dst_p, zrows)

    y2 = pl.pallas_call(
        _tc2_body,
        grid=(NBLK,),
        in_specs=[_parts_spec(H), _row_spec(H), _row_spec(1),
                  _full_spec((H, H)), _full_spec((1, H))],
        out_specs=_row_spec(H),
        out_shape=jax.ShapeDtypeStruct((NACC, H), jnp.float32),
    )(u1, y1, dinv, W1, b1r)

    u2 = _adj_kernel(y2, src_p, dst_p, zrows)

    z = pl.pallas_call(
        _tc3_body,
        grid=(NBLK,),
        in_specs=[_parts_spec(H), _row_spec(H), _row_spec(1),
                  _full_spec((H, H)), _full_spec((1, H)), _row_spec(NHID)],
        out_specs=_row_spec(NHID),
        out_shape=jax.ShapeDtypeStruct((NACC, NHID), jnp.float32),
    )(u2, y2, dinv, W2, b2, eps_p)

    return z[:N]


# speedup vs baseline: 1.1226x; 1.0787x over previous
import functools

import jax
import jax.numpy as jnp
from jax import lax
from jax.experimental import pallas as pl
from jax.experimental.pallas import tpu as pltpu
from jax.experimental.pallas import tpu_sc as plsc

N = 10000
E = 320000
NFEAT = 128
NHID = 64
H = 2 * NHID
MAX_LOGSTD = 10.0

NC = 2
NS = 16
NW = NC * NS
CHUNK = 128
NB = 2
NCH = 80
PH = 2
CPP = NCH // PH
TPE = NCH * CHUNK
EPAD = NW * TPE
NACC = 10112
RPT = NACC // NS
BLK = 128
NBLK = NACC // BLK

_mesh = plsc.VectorSubcoreMesh(core_axis_name="c", subcore_axis_name="s")


@functools.partial(
    pl.kernel,
    out_type=jax.ShapeDtypeStruct((NW, 1, NACC), jnp.float32),
    mesh=_mesh,
    compiler_params=pltpu.CompilerParams(needs_layout_passes=False),
    scratch_types=[
        pltpu.VMEM((NACC,), jnp.float32),
        pltpu.VMEM((NCH, CHUNK), jnp.int32),
    ],
)
def _deg_kernel(dst_hbm, zeros_hbm, out_hbm, hist, didx):
    c = lax.axis_index("c")
    s = lax.axis_index("s")
    wid = s * NC + c
    pltpu.sync_copy(zeros_hbm, hist)
    pltpu.sync_copy(dst_hbm.at[wid], didx)
    ones = jnp.full((16,), 1.0, jnp.float32)

    def body(j, carry):
        for k in range(CHUNK // 16):
            idx = didx[j, pl.ds(k * 16, 16)]
            plsc.addupdate_scatter(hist, [idx], ones)
        return carry

    lax.fori_loop(0, NCH, body, 0)
    pltpu.sync_copy(hist, out_hbm.at[wid, 0])


@functools.partial(
    pl.kernel,
    out_type=jax.ShapeDtypeStruct((NC, NACC, H), jnp.float32),
    mesh=_mesh,
    scratch_types=[
        pltpu.VMEM_SHARED((NACC, H), jnp.float32),
        pltpu.VMEM((CPP, CHUNK), jnp.int32),
        pltpu.VMEM((CPP, CHUNK), jnp.int32),
        pltpu.VMEM((NB, CHUNK, H), jnp.float32),
        pltpu.SemaphoreType.DMA,
        pltpu.SemaphoreType.DMA,
        pltpu.SemaphoreType.DMA,
        pltpu.SemaphoreType.DMA,
    ],
)
def _adj_kernel(y_hbm, src_hbm, dst_hbm, zeros_hbm, out_hbm,
                acc, sidx, didx, rows, sem0, sem1, sem2, sem3):
    sems = (sem0, sem1, sem2, sem3)
    c = lax.axis_index("c")
    s = lax.axis_index("s")
    wid = s * NC + c
    pltpu.sync_copy(zeros_hbm, acc.at[pl.ds(s * RPT, RPT)])
    plsc.subcore_barrier()

    for p in range(PH):
        pltpu.sync_copy(src_hbm.at[wid, pl.ds(p * CPP, CPP)], sidx)
        pltpu.sync_copy(dst_hbm.at[wid, pl.ds(p * CPP, CPP)], didx)

        for b in range(NB):
            pltpu.async_copy(y_hbm.at[sidx.at[b]], rows.at[b], sems[b])

        def body(k, carry):
            for b in range(NB):
                chunk = k * NB + b
                pltpu.make_async_copy(y_hbm.at[sidx.at[chunk]],
                                      rows.at[b], sems[b]).wait()
                pltpu.sync_copy(rows.at[b], acc.at[didx.at[chunk]], add=True)

                @pl.when(chunk + NB < CPP)
                def _():
                    pltpu.async_copy(y_hbm.at[sidx.at[chunk + NB]],
                                     rows.at[b], sems[b])
            return carry

        lax.fori_loop(0, CPP // NB, body, 0)

    plsc.subcore_barrier()
    pltpu.sync_copy(acc.at[pl.ds(s * RPT, RPT)],
                    out_hbm.at[c, pl.ds(s * RPT, RPT)])


def _tc1_body(x_ref, w_ref, b_ref, d_ref, y1_ref, dinv_ref):
    i = pl.program_id(0)
    h0 = jnp.dot(x_ref[...], w_ref[...], preferred_element_type=jnp.float32)
    h0 = h0 + b_ref[...]
    ones_w = jnp.ones((NW, 1), jnp.float32)
    deg = lax.dot_general(d_ref[...], ones_w, (((0,), (0,)), ((), ())),
                          preferred_element_type=jnp.float32) + 1.0
    rows = i * BLK + lax.broadcasted_iota(jnp.int32, (BLK, 1), 0)
    dinv = jnp.where(rows < N, lax.rsqrt(deg), 0.0)
    y1_ref[...] = dinv * h0
    dinv_ref[...] = dinv


def _tc2_body(u_ref, y1_ref, dinv_ref, w_ref, b_ref, y2_ref):
    dinv = dinv_ref[...]
    s1 = dinv * (u_ref[0] + u_ref[1] + y1_ref[...])
    h = jnp.dot(s1, w_ref[...], preferred_element_type=jnp.float32)
    h = jnp.maximum(h + b_ref[...], 0.0)
    y2_ref[...] = dinv * h


def _tc3_body(u_ref, y2_ref, dinv_ref, w_ref, b_ref, eps_ref, z_ref):
    s2 = dinv_ref[...] * (u_ref[0] + u_ref[1] + y2_ref[...])
    o = jnp.dot(s2, w_ref[...], preferred_element_type=jnp.float32)
    o = o + b_ref[...]
    mu = o[:, :NHID]
    ls = jnp.minimum(o[:, NHID:], MAX_LOGSTD)
    z_ref[...] = mu + eps_ref[...] * jnp.exp(ls)


def _row_spec(width):
    return pl.BlockSpec((BLK, width), lambda i: (i, 0))


def _full_spec(shape):
    ndim = len(shape)
    return pl.BlockSpec(shape, lambda i: (0,) * ndim)


def _parts_spec(width):
    return pl.BlockSpec((NC, BLK, width), lambda i: (0, i, 0))


def kernel(x, edge_index, lin_W, lin_b, W1, b1, Wmu, bmu, Wls, bls, eps):
    src = edge_index[0]
    dst = edge_index[1]
    pad_src = jnp.full((EPAD - E,), N, dtype=jnp.int32)
    pad_dst = N + 1 + (jnp.arange(EPAD - E, dtype=jnp.int32) % (NACC - N - 1))
    src_p = jnp.concatenate([src, pad_src]).reshape(NW, NCH, CHUNK)
    dst_p = jnp.concatenate([dst, pad_dst]).reshape(NW, NCH, CHUNK)
    x_p = jnp.pad(x, ((0, NACC - N), (0, 0)))
    eps_p = jnp.pad(eps, ((0, NACC - N), (0, 0)))
    zeros_n = jnp.zeros((NACC,), jnp.float32)
    zrows = jnp.zeros((RPT, H), jnp.float32)
    W2 = jnp.concatenate([Wmu, Wls], axis=1)
    b2 = jnp.concatenate([bmu, bls])[None, :]
    b1r = b1[None, :]
    linbr = lin_b[None, :]

    deg_parts = _deg_kernel(dst_p, zeros_n).reshape(NW, NACC)

    y1, dinv = pl.pallas_call(
        _tc1_body,
        grid=(NBLK,),
        in_specs=[_row_spec(NFEAT), _full_spec((NFEAT, H)),
                  _full_spec((1, H)),
                  pl.BlockSpec((NW, BLK), lambda i: (0, i))],
        out_specs=[_row_spec(H), _row_spec(1)],
        out_shape=[jax.ShapeDtypeStruct((NACC, H), jnp.float32),
                   jax.ShapeDtypeStruct((NACC, 1), jnp.float32)],
    )(x_p, lin_W, linbr, deg_parts)

    u1 = _adj_kernel(y1, src_p, dst_p, zrows)

    y2 = pl.pallas_call(
        _tc2_body,
        grid=(NBLK,),
        in_specs=[_parts_spec(H), _row_spec(H), _row_spec(1),
                  _full_spec((H, H)), _full_spec((1, H))],
        out_specs=_row_spec(H),
        out_shape=jax.ShapeDtypeStruct((NACC, H), jnp.float32),
    )(u1, y1, dinv, W1, b1r)

    u2 = _adj_kernel(y2, src_p, dst_p, zrows)

    z = pl.pallas_call(
        _tc3_body,
        grid=(NBLK,),
        in_specs=[_parts_spec(H), _row_spec(H), _row_spec(1),
                  _full_spec((H, H)), _full_spec((1, H)), _row_spec(NHID)],
        out_specs=_row_spec(NHID),
        out_shape=jax.ShapeDtypeStruct((NACC, NHID), jnp.float32),
    )(u2, y2, dinv, W2, b2, eps_p)

    return z[:N]
